# baseline (speedup 1.0000x reference)
import functools

import jax
import jax.numpy as jnp
from jax import lax
from jax.experimental import pallas as pl
from jax.experimental.pallas import tpu as pltpu
from jax.experimental.pallas import tpu_sc as plsc

BATCH = 16384
LATENT_DIM = 64

_info = plsc.get_sparse_core_info()
_NC, _NS = _info.num_cores, _info.num_subcores
_NW = _NC * _NS
_B_PER_W = BATCH // _NW


@functools.partial(
    pl.kernel,
    mesh=plsc.VectorSubcoreMesh(core_axis_name="c", subcore_axis_name="s"),
    out_type=jax.ShapeDtypeStruct((BATCH, LATENT_DIM), jnp.float32),
    scratch_types=[
        pltpu.VMEM((_B_PER_W,), jnp.int32),
        pltpu.VMEM((_B_PER_W, LATENT_DIM), jnp.float32),
        pltpu.SemaphoreType.DMA,
    ],
    compiler_params=pltpu.CompilerParams(use_tc_tiling_on_sc=False),
)
def _sc_gather(idx_hbm, table_hbm, out_hbm, idx_v, rows_v, sem):
    wid = lax.axis_index("s") * _NC + lax.axis_index("c")
    base = wid * _B_PER_W
    pltpu.sync_copy(idx_hbm.at[pl.ds(base, _B_PER_W)], idx_v)
    pltpu.async_copy(table_hbm.at[idx_v], rows_v, sem).wait()
    pltpu.sync_copy(rows_v, out_hbm.at[pl.ds(base, _B_PER_W)])


def kernel(indices, codes):
    return _sc_gather(indices.astype(jnp.int32), codes)


# speedup vs baseline: 2.0443x; 2.0443x over previous
import functools

import jax
import jax.numpy as jnp
from jax import lax
from jax.experimental import pallas as pl
from jax.experimental.pallas import tpu as pltpu
from jax.experimental.pallas import tpu_sc as plsc

B = 16384
D = 64
V = 1000000
VT = 999936
CH = 512
NCH = VT // CH
_info = plsc.get_sparse_core_info()
_NC, _NS = _info.num_cores, _info.num_subcores
NW = _NC * _NS
CPW = NCH // NW
REM = NCH - CPW * NW

_IOTA = None


@functools.partial(
    pl.kernel,
    mesh=plsc.VectorSubcoreMesh(core_axis_name="c", subcore_axis_name="s"),
    out_type=jax.ShapeDtypeStruct((B + NW, 128), jnp.float32),
    scratch_types=[
        pltpu.VMEM((B,), jnp.int32),
        pltpu.VMEM((B + 16,), jnp.int32),
        pltpu.VMEM((B + 16,), jnp.int32),
        pltpu.VMEM((D, CH), jnp.float32),
        pltpu.VMEM((D, 128), jnp.float32),
        pltpu.VMEM((48,), jnp.int32),
        pltpu.VMEM((48,), jnp.int32),
        pltpu.VMEM((128, 128), jnp.float32),
        pltpu.VMEM((2, 64), jnp.int32),
        pltpu.SemaphoreType.DMA,
        pltpu.SemaphoreType.DMA,
    ],
    compiler_params=pltpu.CompilerParams(
        use_tc_tiling_on_sc=True, needs_layout_passes=False),
)
def _sc_lookup(idx_hbm, tabT_hbm, tailT_hbm, out_hbm,
               idx_v, i_sel, b_sel, buf_v, tail_v, ring_i, ring_b,
               stage_v, b_stage, sem0, sem1):
    wid = lax.axis_index("s") * _NC + lax.axis_index("c")
    dummy = B + wid
    iota16 = lax.iota(jnp.int32, 16)
    d16 = [iota16 + 16 * g for g in range(4)]

    chlo = wid * CPW + jnp.minimum(wid, REM)
    nch = CPW + jnp.where(wid < REM, 1, 0)
    lo_e = chlo * CH
    hi_e = jnp.where(wid == NW - 1, V, (chlo + nch) * CH)

    for h in range(2):
        for q in range(4):
            b_stage[h, pl.ds(q * 16, 16)] = jnp.full((16,), dummy, jnp.int32)

    pltpu.sync_copy(idx_hbm, idx_v)

    def sel_body(p, n):
        v = idx_v[pl.ds(p * 16, 16)]
        m = (v >= lo_e) & (v < hi_e)
        cnt = plsc.all_reduce_population_count(m)[0]
        plsc.store_compressed(i_sel.at[pl.ds(n, 16)], v, mask=m)
        plsc.store_compressed(b_sel.at[pl.ds(n, 16)], iota16 + p * 16, mask=m)
        return n + cnt

    n = lax.fori_loop(0, B // 16, sel_body, jnp.int32(0))
    npacks = (n + 15) // 16

    def fire_half(h):
        sem = sem0 if h == 0 else sem1
        pltpu.async_copy(
            stage_v.at[pl.ds(h * 64, 64)],
            out_hbm.at[b_stage.at[h]], sem)

    def wait_half(h):
        sem = sem0 if h == 0 else sem1
        pltpu.make_async_copy(
            stage_v.at[pl.ds(h * 64, 64)],
            out_hbm.at[b_stage.at[h]], sem).wait()

    def flush(h, f):
        @pl.when(f >= 1)
        def _():
            wait_half(1 - h)

        for q in range(4):
            b_stage[1 - h, pl.ds(q * 16, 16)] = jnp.full(
                (16,), dummy, jnp.int32)
        fire_half(h)

    def maybe_flush(sp, f, changed):
        do = changed & (lax.rem(sp, 64) == 0)
        par = lax.rem(sp // 64, 2)

        @pl.when(do & (par == 1))
        def _():
            flush(0, f)

        @pl.when(do & (par == 0))
        def _():
            flush(1, f)

        return f + jnp.where(do, 1, 0)

    def drain(buf, m, sp):
        loc = jnp.where(m, ring_i[pl.ds(0, 16)], 0)
        b16 = jnp.where(m, ring_b[pl.ds(0, 16)], dummy)
        srow = lax.rem(sp, 128)
        h = srow // 64
        plsc.store_scatter(b_stage, [jnp.full((16,), h, jnp.int32),
                                     lax.rem(srow, 64) + iota16], b16)
        for k in range(16):
            lk = loc[k]
            ck = jnp.full((16,), lk, jnp.int32)
            for g in range(4):
                vals = plsc.load_gather(buf, [d16[g], ck])
                stage_v[srow + k, pl.ds(g * 16, 16)] = vals

    def process_range(buf, clo, width, carry):
        sp0, f0 = carry

        def pr_body(p, c):
            sp, rc, f = c
            v = i_sel[pl.ds(p * 16, 16)]
            b = b_sel[pl.ds(p * 16, 16)]
            valid = (iota16 + p * 16) < n
            m = valid & (v >= clo) & (v < clo + width)
            cnt = plsc.all_reduce_population_count(m)[0]
            plsc.store_compressed(ring_i.at[pl.ds(rc, 16)], v - clo, mask=m)
            plsc.store_compressed(ring_b.at[pl.ds(rc, 16)], b, mask=m)
            rc = rc + cnt
            full = rc >= 16

            @pl.when(full)
            def _():
                drain(buf, iota16 < 16, sp)
                t_i = ring_i[pl.ds(16, 16)]
                t_b = ring_b[pl.ds(16, 16)]
                ring_i[pl.ds(0, 16)] = t_i
                ring_b[pl.ds(0, 16)] = t_b

            sp = sp + jnp.where(full, 16, 0)
            rc = rc - jnp.where(full, 16, 0)
            f = maybe_flush(sp, f, full)
            return (sp, rc, f)

        sp, rc, f = lax.fori_loop(0, npacks, pr_body, (sp0, jnp.int32(0), f0))

        part = rc > 0

        @pl.when(part)
        def _():
            drain(buf, iota16 < rc, sp)

        sp = sp + jnp.where(part, 16, 0)
        f = maybe_flush(sp, f, part)
        return (sp, f)

    def chunk_body(c, carry):
        clo = (chlo + c) * CH
        pltpu.sync_copy(
            tabT_hbm.at[:, pl.ds(pl.multiple_of(clo, 128), CH)], buf_v)
        return process_range(buf_v, clo, CH, carry)

    carry = lax.fori_loop(0, nch, chunk_body, (jnp.int32(0), jnp.int32(0)))

    def tail_fn(c):
        pltpu.sync_copy(tailT_hbm, tail_v)
        return process_range(tail_v, jnp.int32(VT), V - VT, c)

    sp, f = lax.cond(wid == NW - 1, tail_fn, lambda c: c, carry)

    for h in (0, 1):
        @pl.when((f >= 1) & (lax.rem(f - 1, 2) == h))
        def _(h=h):
            wait_half(h)

    pend = lax.rem(sp, 64) != 0
    par = lax.rem(sp // 64, 2)
    for h in (0, 1):
        @pl.when(pend & (par == h))
        def _(h=h):
            fire_half(h)
            wait_half(h)


def kernel(indices, codes):
    idx = indices.astype(jnp.int32)
    tabT = codes.T
    tailT = jnp.zeros((D, 128), jnp.float32).at[:, :V - VT].set(
        codes[VT:].T)
    wide = _sc_lookup(idx, tabT, tailT)
    return wide[:B, :D]


# speedup vs baseline: 3.0241x; 1.4793x over previous
import functools

import jax
import jax.numpy as jnp
from jax import lax
from jax.experimental import pallas as pl
from jax.experimental.pallas import tpu as pltpu
from jax.experimental.pallas import tpu_sc as plsc

B = 16384
D = 64
V = 1000000
VT = 999936
CH = 512
NCH = VT // CH
_info = plsc.get_sparse_core_info()
_NC, _NS = _info.num_cores, _info.num_subcores
NW = _NC * _NS
CPW = NCH // NW
REM = NCH - CPW * NW

_IOTA = None


@functools.partial(
    pl.kernel,
    mesh=plsc.VectorSubcoreMesh(core_axis_name="c", subcore_axis_name="s"),
    out_type=jax.ShapeDtypeStruct((B + NW, 128), jnp.float32),
    scratch_types=[
        pltpu.VMEM((B,), jnp.int32),
        pltpu.VMEM((B + 16,), jnp.int32),
        pltpu.VMEM((D, CH), jnp.float32),
        pltpu.VMEM((D, CH), jnp.float32),
        pltpu.VMEM((D, 128), jnp.float32),
        pltpu.VMEM((48,), jnp.int32),
        pltpu.VMEM((48,), jnp.int32),
        pltpu.VMEM((128, 128), jnp.float32),
        pltpu.VMEM((2, 64), jnp.int32),
        pltpu.SemaphoreType.DMA,
        pltpu.SemaphoreType.DMA,
        pltpu.SemaphoreType.DMA,
        pltpu.SemaphoreType.DMA,
    ],
    compiler_params=pltpu.CompilerParams(
        use_tc_tiling_on_sc=True, needs_layout_passes=False),
)
def _sc_lookup(idx_hbm, tabT_hbm, tailT_hbm, out_hbm,
               idx_v, b_sel, cbuf0, cbuf1, tail_v, ring_i, ring_b,
               stage_v, b_stage, sem0, sem1, sem2, sem3):
    wid = lax.axis_index("s") * _NC + lax.axis_index("c")
    dummy = B + wid
    iota16 = lax.iota(jnp.int32, 16)
    d16 = [iota16 + 16 * g for g in range(4)]

    chlo = wid * CPW + jnp.minimum(wid, REM)
    nch = CPW + jnp.where(wid < REM, 1, 0)
    lo_e = chlo * CH
    hi_e = jnp.where(wid == NW - 1, V, (chlo + nch) * CH)

    for h in range(2):
        for q in range(4):
            b_stage[h, pl.ds(q * 16, 16)] = jnp.full((16,), dummy, jnp.int32)

    pltpu.sync_copy(idx_hbm, idx_v)

    def sel_body(p, n):
        v = idx_v[pl.ds(p * 16, 16)]
        m = (v >= lo_e) & (v < hi_e)
        cnt = plsc.all_reduce_population_count(m)[0]
        plsc.store_compressed(b_sel.at[pl.ds(n, 16)], iota16 + p * 16, mask=m)
        return n + cnt

    n = lax.fori_loop(0, B // 16, sel_body, jnp.int32(0))
    npacks = (n + 15) // 16

    def fire_half(h):
        sem = sem0 if h == 0 else sem1
        pltpu.async_copy(
            stage_v.at[pl.ds(h * 64, 64)],
            out_hbm.at[b_stage.at[h]], sem)

    def wait_half(h):
        sem = sem0 if h == 0 else sem1
        pltpu.make_async_copy(
            stage_v.at[pl.ds(h * 64, 64)],
            out_hbm.at[b_stage.at[h]], sem).wait()

    def flush(h, f):
        @pl.when(f >= 1)
        def _():
            wait_half(1 - h)

        for q in range(4):
            b_stage[1 - h, pl.ds(q * 16, 16)] = jnp.full(
                (16,), dummy, jnp.int32)
        fire_half(h)

    def maybe_flush(sp, f, changed):
        do = changed & (lax.rem(sp, 64) == 0)
        par = lax.rem(sp // 64, 2)

        @pl.when(do & (par == 1))
        def _():
            flush(0, f)

        @pl.when(do & (par == 0))
        def _():
            flush(1, f)

        return f + jnp.where(do, 1, 0)

    def drain(buf, m, sp):
        loc = jnp.where(m, ring_i[pl.ds(0, 16)], 0)
        b16 = jnp.where(m, ring_b[pl.ds(0, 16)], dummy)
        srow = lax.rem(sp, 128)
        h = srow // 64
        plsc.store_scatter(b_stage, [jnp.full((16,), h, jnp.int32),
                                     lax.rem(srow, 64) + iota16], b16)
        for k in range(16):
            lk = loc[k]
            ck = jnp.full((16,), lk, jnp.int32)
            for g in range(4):
                vals = plsc.load_gather(buf, [d16[g], ck])
                stage_v[srow + k, pl.ds(g * 16, 16)] = vals

    def process_range(buf, clo, width, carry):
        sp0, f0 = carry

        def pr_body(p, c):
            sp, rc, f = c
            valid = (iota16 + p * 16) < n
            b = jnp.where(valid, b_sel[pl.ds(p * 16, 16)], 0)
            v = plsc.load_gather(idx_v, [b])
            m = valid & (v >= clo) & (v < clo + width)
            cnt = plsc.all_reduce_population_count(m)[0]
            plsc.store_compressed(ring_i.at[pl.ds(rc, 16)], v - clo, mask=m)
            plsc.store_compressed(ring_b.at[pl.ds(rc, 16)], b, mask=m)
            rc = rc + cnt
            full = rc >= 16

            @pl.when(full)
            def _():
                drain(buf, iota16 < 16, sp)
                t_i = ring_i[pl.ds(16, 16)]
                t_b = ring_b[pl.ds(16, 16)]
                ring_i[pl.ds(0, 16)] = t_i
                ring_b[pl.ds(0, 16)] = t_b

            sp = sp + jnp.where(full, 16, 0)
            rc = rc - jnp.where(full, 16, 0)
            f = maybe_flush(sp, f, full)
            return (sp, rc, f)

        sp, rc, f = lax.fori_loop(0, npacks, pr_body, (sp0, jnp.int32(0), f0))

        part = rc > 0

        @pl.when(part)
        def _():
            drain(buf, iota16 < rc, sp)

        sp = sp + jnp.where(part, 16, 0)
        f = maybe_flush(sp, f, part)
        return (sp, f)

    def fire_chunk(c, buf, sem):
        clo = pl.multiple_of((chlo + c) * CH, 128)
        pltpu.async_copy(tabT_hbm.at[:, pl.ds(clo, CH)], buf, sem)

    def wait_chunk(buf, sem):
        pltpu.make_async_copy(
            tabT_hbm.at[:, pl.ds(0, CH)], buf, sem).wait()

    def chunk_step(c, buf, sem_my, buf_o, sem_o):
        def fn(cr):
            wait_chunk(buf, sem_my)

            @pl.when(c + 1 < nch)
            def _():
                fire_chunk(c + 1, buf_o, sem_o)

            return process_range(buf, (chlo + c) * CH, CH, cr)

        return fn

    def chunk_body(c, carry):
        return lax.cond(
            lax.rem(c, 2) == 0,
            chunk_step(c, cbuf0, sem2, cbuf1, sem3),
            chunk_step(c, cbuf1, sem3, cbuf0, sem2),
            carry)

    fire_chunk(0, cbuf0, sem2)
    carry = lax.fori_loop(0, nch, chunk_body, (jnp.int32(0), jnp.int32(0)))

    def tail_fn(c):
        pltpu.sync_copy(tailT_hbm, tail_v)
        return process_range(tail_v, jnp.int32(VT), V - VT, c)

    sp, f = lax.cond(wid == NW - 1, tail_fn, lambda c: c, carry)

    for h in (0, 1):
        @pl.when((f >= 1) & (lax.rem(f - 1, 2) == h))
        def _(h=h):
            wait_half(h)

    pend = lax.rem(sp, 64) != 0
    par = lax.rem(sp // 64, 2)
    for h in (0, 1):
        @pl.when(pend & (par == h))
        def _(h=h):
            fire_half(h)
            wait_half(h)


def kernel(indices, codes):
    idx = indices.astype(jnp.int32)
    tabT = codes.T
    tailT = jnp.zeros((D, 128), jnp.float32).at[:, :V - VT].set(
        codes[VT:].T)
    wide = _sc_lookup(idx, tabT, tailT)
    return wide[:B, :D]
